# baseline (speedup 1.0000x reference)
import functools

import jax
import jax.numpy as jnp
from jax import lax
from jax.experimental import pallas as pl

B = 1024
D = 128
N_ITEMS = 100000
BLK = 2048
N_PAD = 100352
K = 100


def _scores_body(u_ref, t_ref, out_ref):
    out_ref[...] = lax.dot_general(
        u_ref[...], t_ref[...], (((1,), (1,)), ((), ())),
        preferred_element_type=jnp.float32)


def _scores(u, item_table):
    grid = (N_PAD // BLK,)
    return pl.pallas_call(
        _scores_body,
        grid=grid,
        in_specs=[
            pl.BlockSpec((B, D), lambda i: (0, 0)),
            pl.BlockSpec((BLK, D), lambda i: (i, 0)),
        ],
        out_specs=pl.BlockSpec((B, BLK), lambda i: (0, i)),
        out_shape=jax.ShapeDtypeStruct((B, N_PAD), jnp.float32),
    )(u, item_table)


def kernel(user_id, user_features, user_history, user_id_table, item_id_table, Wf, bf, Wt, bt):
    user_history_embedding = jnp.take(item_id_table, user_history, axis=0)
    user_history_summary = user_history_embedding.mean(axis=1)
    user_id_embedding = jnp.take(user_id_table, user_id, axis=0)
    user_features_embedding = user_features @ Wf.T + bf
    user_tower_input = jnp.concatenate(
        [user_id_embedding, user_features_embedding, user_history_summary], axis=1)
    user_embedding = user_tower_input @ Wt.T + bt
    scores = _scores(user_embedding, item_id_table)[:, :N_ITEMS]
    top_vals, top_items = jax.lax.top_k(scores, K)
    return top_items


# speedup vs baseline: 6.2825x; 6.2825x over previous
import functools
import math

import jax
import jax.numpy as jnp
from jax import lax
from jax.experimental import pallas as pl

BLK = 2048
CH = 128
K = 100


def _scores_body(iv, n_items_pad, u_ref, t_ref, s_ref, m_ref):
    i = pl.program_id(0)
    s = lax.dot_general(
        u_ref[...], t_ref[...], (((1,), (1,)), ((), ())),
        preferred_element_type=jnp.float32)
    idx = i * BLK + lax.broadcasted_iota(jnp.int32, s.shape, 1)
    s = jnp.where(idx < iv, s, -jnp.inf)
    s_ref[...] = s
    b = s.shape[0]
    m_ref[0] = s.reshape(b, BLK // CH, CH).max(axis=2)


def _scores_and_chunkmax(u, item_table):
    b, d = u.shape
    iv = item_table.shape[0]
    n_pad = math.ceil(iv / BLK) * BLK
    if n_pad != iv:
        item_table = jnp.pad(item_table, ((0, n_pad - iv), (0, 0)))
    grid = (n_pad // BLK,)
    return pl.pallas_call(
        functools.partial(_scores_body, iv, n_pad),
        grid=grid,
        in_specs=[
            pl.BlockSpec((b, d), lambda i: (0, 0)),
            pl.BlockSpec((BLK, d), lambda i: (i, 0)),
        ],
        out_specs=[
            pl.BlockSpec((b, BLK), lambda i: (0, i)),
            pl.BlockSpec((1, b, BLK // CH), lambda i: (i, 0, 0)),
        ],
        out_shape=[
            jax.ShapeDtypeStruct((b, n_pad), jnp.float32),
            jax.ShapeDtypeStruct((n_pad // BLK, b, BLK // CH), jnp.float32),
        ],
    )(u, item_table)


def kernel(user_id, user_features, user_history, user_id_table, item_id_table,
           Wf, bf, Wt, bt):
    user_history_embedding = jnp.take(item_id_table, user_history, axis=0)
    user_history_summary = user_history_embedding.mean(axis=1)
    user_id_embedding = jnp.take(user_id_table, user_id, axis=0)
    user_features_embedding = user_features @ Wf.T + bf
    user_tower_input = jnp.concatenate(
        [user_id_embedding, user_features_embedding, user_history_summary],
        axis=1)
    user_embedding = user_tower_input @ Wt.T + bt

    scores, cmax3 = _scores_and_chunkmax(user_embedding, item_id_table)
    b, n_pad = scores.shape
    nchunk = n_pad // CH
    cmax = cmax3.transpose(1, 0, 2).reshape(b, nchunk)

    kc = min(K, nchunk)
    _, chunk_idx = lax.top_k(cmax, kc)
    cand = jnp.take_along_axis(
        scores.reshape(b, nchunk, CH), chunk_idx[:, :, None], axis=1)
    _, pos = lax.top_k(cand.reshape(b, kc * CH), K)
    chunk_of = jnp.take_along_axis(chunk_idx, pos // CH, axis=1)
    top_items = chunk_of * CH + pos % CH
    return top_items


# speedup vs baseline: 8.8877x; 1.4147x over previous
import functools
import math

import jax
import jax.numpy as jnp
from jax import lax
from jax.experimental import pallas as pl

BLK = 2048
GRP = BLK // 128
K = 100


def _scores_body(iv, u_ref, t_ref, s_ref, m_ref):
    i = pl.program_id(0)
    s = lax.dot_general(
        u_ref[...], t_ref[...], (((1,), (1,)), ((), ())),
        preferred_element_type=jnp.float32)
    idx = i * BLK + lax.broadcasted_iota(jnp.int32, s.shape, 1)
    s = jnp.where(idx < iv, s, -jnp.inf)
    s_ref[...] = s
    b = s.shape[0]
    m_ref[0] = s.reshape(b, GRP, 128).max(axis=1)


def _scores_and_groupmax(u, item_table):
    b, d = u.shape
    iv = item_table.shape[0]
    n_pad = math.ceil(iv / BLK) * BLK
    if n_pad != iv:
        item_table = jnp.pad(item_table, ((0, n_pad - iv), (0, 0)))
    grid = (n_pad // BLK,)
    return pl.pallas_call(
        functools.partial(_scores_body, iv),
        grid=grid,
        in_specs=[
            pl.BlockSpec((b, d), lambda i: (0, 0)),
            pl.BlockSpec((BLK, d), lambda i: (i, 0)),
        ],
        out_specs=[
            pl.BlockSpec((b, BLK), lambda i: (0, i)),
            pl.BlockSpec((1, b, 128), lambda i: (i, 0, 0)),
        ],
        out_shape=[
            jax.ShapeDtypeStruct((b, n_pad), jnp.float32),
            jax.ShapeDtypeStruct((n_pad // BLK, b, 128), jnp.float32),
        ],
    )(u, item_table)


def kernel(user_id, user_features, user_history, user_id_table, item_id_table,
           Wf, bf, Wt, bt):
    user_history_embedding = jnp.take(item_id_table, user_history, axis=0)
    user_history_summary = user_history_embedding.mean(axis=1)
    user_id_embedding = jnp.take(user_id_table, user_id, axis=0)
    user_features_embedding = user_features @ Wf.T + bf
    user_tower_input = jnp.concatenate(
        [user_id_embedding, user_features_embedding, user_history_summary],
        axis=1)
    user_embedding = user_tower_input @ Wt.T + bt

    scores, gmax3 = _scores_and_groupmax(user_embedding, item_id_table)
    b, n_pad = scores.shape
    ngrp = n_pad // GRP
    gmax = gmax3.transpose(1, 0, 2).reshape(b, ngrp)

    kg = min(K, ngrp)
    _, grp_idx = lax.top_k(gmax, kg)
    g = jnp.arange(GRP, dtype=grp_idx.dtype)
    p = (grp_idx[:, :, None] // 128 * BLK
         + grp_idx[:, :, None] % 128
         + g[None, None, :] * 128).reshape(b, kg * GRP)
    cand = jnp.take_along_axis(scores, p, axis=1)
    _, pos = lax.top_k(cand, K)
    top_items = jnp.take_along_axis(p, pos, axis=1)
    return top_items


# speedup vs baseline: 18.5418x; 2.0862x over previous
import functools
import math

import jax
import jax.numpy as jnp
from jax import lax
from jax.experimental import pallas as pl

BLK = 2048
K = 100


def _scores_body(iv, u_ref, t_ref, s_ref, m_ref):
    i = pl.program_id(0)
    s = lax.dot_general(
        u_ref[...], t_ref[...], (((1,), (1,)), ((), ())),
        preferred_element_type=jnp.float32)
    idx = i * BLK + lax.broadcasted_iota(jnp.int32, s.shape, 1)
    s = jnp.where(idx < iv, s, -jnp.inf)
    s_ref[...] = s
    b = s.shape[0]
    m_ref[0] = s.reshape(b, 4, 4, 128).max(axis=2).reshape(b, 512)


def _scores_and_groupmax(u, item_table):
    b, d = u.shape
    iv = item_table.shape[0]
    n_pad = math.ceil(iv / BLK) * BLK
    if n_pad != iv:
        item_table = jnp.pad(item_table, ((0, n_pad - iv), (0, 0)))
    grid = (n_pad // BLK,)
    return pl.pallas_call(
        functools.partial(_scores_body, iv),
        grid=grid,
        in_specs=[
            pl.BlockSpec((b, d), lambda i: (0, 0)),
            pl.BlockSpec((BLK, d), lambda i: (i, 0)),
        ],
        out_specs=[
            pl.BlockSpec((b, BLK), lambda i: (0, i)),
            pl.BlockSpec((1, b, 512), lambda i: (i, 0, 0)),
        ],
        out_shape=[
            jax.ShapeDtypeStruct((b, n_pad), jnp.float32),
            jax.ShapeDtypeStruct((n_pad // BLK, b, 512), jnp.float32),
        ],
    )(u, item_table)


def _refine(vals, child_ids, k):
    b, n, r = child_ids.shape
    flat = child_ids.reshape(b, n * r)
    cand = jnp.take_along_axis(vals, flat, axis=1)
    _, pos = lax.top_k(cand, min(k, n * r))
    return jnp.take_along_axis(flat, pos, axis=1)


def kernel(user_id, user_features, user_history, user_id_table, item_id_table,
           Wf, bf, Wt, bt):
    user_history_embedding = jnp.take(item_id_table, user_history, axis=0)
    user_history_summary = user_history_embedding.mean(axis=1)
    user_id_embedding = jnp.take(user_id_table, user_id, axis=0)
    user_features_embedding = user_features @ Wf.T + bf
    user_tower_input = jnp.concatenate(
        [user_id_embedding, user_features_embedding, user_history_summary],
        axis=1)
    user_embedding = user_tower_input @ Wt.T + bt

    scores, sub3 = _scores_and_groupmax(user_embedding, item_id_table)
    b, n_pad = scores.shape
    nblk = n_pad // BLK

    st = sub3.transpose(1, 0, 2)
    a4 = st.reshape(b, nblk * 512)
    a16 = st.reshape(b, nblk, 4, 128).max(axis=2)
    a64 = a16.reshape(b, nblk, 32, 4).max(axis=3)
    a256 = a64.reshape(b, nblk, 8, 4).max(axis=3)
    a16 = a16.reshape(b, nblk * 128)
    a64 = a64.reshape(b, nblk * 32)
    a256 = a256.reshape(b, nblk * 8)

    d4 = jnp.arange(4, dtype=jnp.int32)
    kz = min(K, nblk * 8)
    _, z = lax.top_k(a256, kz)
    y = _refine(a64, (z // 8 * 32 + z % 8 * 4)[:, :, None] + d4, K)
    g = _refine(a16, (y // 32 * 128 + y % 32 * 4)[:, :, None] + d4, K)
    s4 = _refine(a4, (g // 128 * 512 + g % 128)[:, :, None] + d4 * 128, K)
    i_ = s4 // 512
    m_ = s4 % 512 // 128
    x_ = s4 % 128
    items = (i_ * 2048 + m_ * 512 + x_)[:, :, None] + d4 * 128
    top_items = _refine(scores, items, K)
    return top_items
